# baseline (speedup 1.0000x reference)
import functools

import jax
import jax.numpy as jnp
from jax import lax
from jax.experimental import pallas as pl
from jax.experimental.pallas import tpu as pltpu
from jax.experimental.pallas import tpu_sc as plsc

BATCH = 4096
HIST = 50
D_MODEL = 128
NUM_WORKERS = 32
BPW = BATCH // NUM_WORKERS
GROUP = 2
STEPS = HIST // GROUP
NBUF = 3
LAG = 1

_mesh = plsc.VectorSubcoreMesh(core_axis_name="c", subcore_axis_name="s")


@functools.partial(
    pl.kernel,
    mesh=_mesh,
    out_type=jax.ShapeDtypeStruct((HIST, BATCH, D_MODEL), jnp.float32),
    scratch_types=[
        pltpu.VMEM((HIST, BPW), jnp.int32),
        pltpu.VMEM((NBUF, GROUP, BPW, D_MODEL), jnp.float32),
        pltpu.SemaphoreType.DMA((NBUF,)),
        pltpu.SemaphoreType.DMA((NBUF,)),
    ],
)
def _gather_kernel(table_hbm, idx_hbm, out_hbm, idx_v, rows_v, gsem, ssem):
    wid = lax.axis_index("s") * 2 + lax.axis_index("c")
    batch0 = wid * BPW
    pltpu.sync_copy(idx_hbm.at[:, pl.ds(batch0, BPW)], idx_v)

    def body(i, carry):
        b = lax.rem(i, NBUF)

        @pl.when(i >= NBUF)
        def _():
            pltpu.make_async_copy(
                rows_v.at[b],
                out_hbm.at[pl.ds((i - NBUF) * GROUP, GROUP),
                           pl.ds(batch0, BPW)],
                ssem.at[b]).wait()

        @pl.when(i < STEPS)
        def _():
            for k in range(GROUP):
                pltpu.async_copy(
                    table_hbm.at[idx_v.at[i * GROUP + k]],
                    rows_v.at[b, k], gsem.at[b])

        h = i - LAG
        bh = lax.rem(i + (NBUF - LAG), NBUF)

        @pl.when((i >= LAG) & (h < STEPS))
        def _():
            for k in range(GROUP):
                pltpu.make_async_copy(
                    table_hbm.at[idx_v.at[h * GROUP + k]],
                    rows_v.at[bh, k], gsem.at[bh]).wait()
            pltpu.async_copy(
                rows_v.at[bh],
                out_hbm.at[pl.ds(h * GROUP, GROUP), pl.ds(batch0, BPW)],
                ssem.at[bh])

        return carry

    lax.fori_loop(0, STEPS + NBUF, body, None)


def kernel(key, embedding_weight):
    idx_t = key.astype(jnp.int32).T
    out_t = _gather_kernel(embedding_weight, idx_t)
    return jnp.transpose(out_t, (1, 0, 2))


# speedup vs baseline: 1.0251x; 1.0251x over previous
import functools

import jax
import jax.numpy as jnp
from jax import lax
from jax.experimental import pallas as pl
from jax.experimental.pallas import tpu as pltpu
from jax.experimental.pallas import tpu_sc as plsc

BATCH = 4096
HIST = 50
D_MODEL = 128
NUM_WORKERS = 32
BPW = BATCH // NUM_WORKERS
NBUF = 7
LAG = 5

_mesh = plsc.VectorSubcoreMesh(core_axis_name="c", subcore_axis_name="s")


@functools.partial(
    pl.kernel,
    mesh=_mesh,
    out_type=jax.ShapeDtypeStruct((HIST, BATCH, D_MODEL), jnp.float32),
    scratch_types=[
        pltpu.VMEM((HIST, BPW), jnp.int32),
        pltpu.VMEM((NBUF, BPW, D_MODEL), jnp.float32),
        pltpu.SemaphoreType.DMA((NBUF,)),
        pltpu.SemaphoreType.DMA((NBUF,)),
    ],
)
def _gather_kernel(table_hbm, idx_hbm, out_hbm, idx_v, rows_v, gsem, ssem):
    wid = lax.axis_index("s") * 2 + lax.axis_index("c")
    batch0 = wid * BPW
    pltpu.sync_copy(idx_hbm.at[:, pl.ds(batch0, BPW)], idx_v)

    def body(i, carry):
        b = lax.rem(i, NBUF)

        @pl.when(i >= NBUF)
        def _():
            pltpu.make_async_copy(
                rows_v.at[b],
                out_hbm.at[i - NBUF, pl.ds(batch0, BPW)],
                ssem.at[b]).wait()

        @pl.when(i < HIST)
        def _():
            pltpu.async_copy(
                table_hbm.at[idx_v.at[i]], rows_v.at[b], gsem.at[b])

        h = i - LAG
        bh = lax.rem(i + (NBUF - LAG), NBUF)

        @pl.when((i >= LAG) & (h < HIST))
        def _():
            pltpu.make_async_copy(
                table_hbm.at[idx_v.at[h]], rows_v.at[bh], gsem.at[bh]).wait()
            pltpu.async_copy(
                rows_v.at[bh],
                out_hbm.at[h, pl.ds(batch0, BPW)],
                ssem.at[bh])

        return carry

    lax.fori_loop(0, HIST + NBUF, body, None)


def kernel(key, embedding_weight):
    idx_t = key.astype(jnp.int32).T
    out_t = _gather_kernel(embedding_weight, idx_t)
    return jnp.transpose(out_t, (1, 0, 2))
